# Initial kernel scaffold; baseline (speedup 1.0000x reference)
import jax
import jax.numpy as jnp
from jax.experimental import pallas as pl


def kernel(x, edge_index, batch, eigens, emb, W, att_src, att_dst, bias, W1, b1, W2, b2):
    raise NotImplementedError("write your pallas kernel here")


# speedup vs baseline: 16.5868x; 16.5868x over previous
import jax
import jax.numpy as jnp
from jax import lax
from jax.experimental import pallas as pl
from jax.experimental.pallas import tpu as pltpu
from jax.experimental.pallas import tpu_sc as plsc

N = 10000
E = 320000
H = 128
G = 64
V = 28
NC = 2
NS = 16
NW = NC * NS
EPW = E // NW
K = 80
CH = EPW // K
L = 16
NRC = N // L
RC = (NRC + NS - 1) // NS
EPT = E // NS
CH2 = EPT // K
NH = N // 2
NHP = 5008
NRC2 = NHP // L
RC2 = (NRC2 + NS - 1) // NS
F32 = jnp.float32
I32 = jnp.int32

_HIGH = lax.Precision.HIGHEST
BF16 = jnp.bfloat16


def _split(a):
    hi = a.astype(BF16)
    lo = (a - hi.astype(F32)).astype(BF16)
    return hi, lo


def _dot_f32(a, b):
    return jnp.dot(a, b, preferred_element_type=F32)


def _dot_onehot(oh, b):
    bh, bl = _split(b)
    ohb = oh.astype(BF16)

    def d(x, y):
        return jnp.dot(x, y, preferred_element_type=F32)

    return d(ohb, bh) + d(ohb, bl)

_mesh = plsc.VectorSubcoreMesh(
    core_axis_name="c", subcore_axis_name="s", num_cores=NC, num_subcores=NS)
_sc_params = pltpu.CompilerParams(needs_layout_passes=False)


def _p1_body(src_h, dst_h, as_h, ad_h, m_h, zeros_h, ex_h, dp0_h, dp1_h,
             asv, adv, srcv, dstv, exv, mv, dsh):
    c = lax.axis_index("c")
    s = lax.axis_index("s")
    wid = s * NC + c
    base = wid * EPW

    pltpu.sync_copy(as_h, asv)
    pltpu.sync_copy(ad_h, adv)
    pltpu.sync_copy(src_h.at[pl.ds(base, EPW)], srcv)
    pltpu.sync_copy(dst_h.at[pl.ds(base, EPW)], dstv)

    @pl.when(s == 0)
    def _():
        pltpu.sync_copy(zeros_h, dsh)

    pltpu.sync_copy(m_h.at[0, pl.ds(0, L)], mv)
    m = mv[...]

    def step(i, _):
        sidx = srcv[pl.ds(i * L, L)]
        didx = dstv[pl.ds(i * L, L)]
        t = plsc.load_gather(asv, [sidx]) + plsc.load_gather(adv, [didx])
        e = jnp.maximum(t, 0.2 * t)
        exv[pl.ds(i * L, L)] = jnp.exp(e - m)
        return 0

    lax.fori_loop(0, EPW // L, step, 0)

    plsc.subcore_barrier()
    pltpu.sync_copy(exv, dsh.at[dstv], add=True)
    pltpu.sync_copy(exv, ex_h.at[pl.ds(base, EPW)])
    plsc.subcore_barrier()

    @pl.when((s == 0) & (c == 0))
    def _():
        pltpu.sync_copy(dsh, dp0_h)

    @pl.when((s == 0) & (c == 1))
    def _():
        pltpu.sync_copy(dsh, dp1_h)


_edge_softmax = pl.kernel(
    _p1_body,
    out_type=[jax.ShapeDtypeStruct((E,), F32),
              jax.ShapeDtypeStruct((N,), F32),
              jax.ShapeDtypeStruct((N,), F32)],
    mesh=_mesh,
    compiler_params=_sc_params,
    scratch_types=[
        pltpu.VMEM((N,), F32),
        pltpu.VMEM((N,), F32),
        pltpu.VMEM((EPW,), I32),
        pltpu.VMEM((EPW,), I32),
        pltpu.VMEM((EPW,), F32),
        pltpu.VMEM((L,), F32),
        pltpu.VMEM_SHARED((N,), F32),
    ],
)


def _p2_body(hw_h, src_h, dst3_h, ex_h, dp0_h, dp1_h, acc_h,
             srcv, dstv, exv, sadj, dadj, rows, zbuf, tbuf, dp0v, dp1v, invv,
             acc_sh, sem):
    c = lax.axis_index("c")
    s = lax.axis_index("s")
    base = s * EPT
    lo = c * NH

    pltpu.sync_copy(src_h.at[pl.ds(base, EPT)], srcv)
    pltpu.sync_copy(dst3_h.at[pl.ds(base, EPT)], dstv)
    pltpu.sync_copy(ex_h.at[pl.ds(base, EPT)], exv)
    pltpu.sync_copy(dp0_h.at[pl.ds(lo, NH)], dp0v.at[pl.ds(0, NH)])
    pltpu.sync_copy(dp1_h.at[pl.ds(lo, NH)], dp1v.at[pl.ds(0, NH)])

    def invstep(i, _):
        d = dp0v[pl.ds(i * L, L)] + dp1v[pl.ds(i * L, L)]
        invv[pl.ds(i * L, L)] = 1.0 / (d + 1e-16)
        return 0

    lax.fori_loop(0, NRC2, invstep, 0)

    zv = jnp.zeros((L,), F32)
    for r in range(L):
        for l in range(H // L):
            zbuf[r, pl.ds(l * L, L)] = zv

    def zcpy(k, _):
        cid = s + k * NS

        @pl.when(cid < NRC2)
        def _():
            pltpu.sync_copy(zbuf, acc_sh.at[pl.ds(cid * L, L)])
        return 0

    lax.fori_loop(0, RC2, zcpy, 0)
    plsc.subcore_barrier()

    def chunk(i, _):
        for v in range(K // L):
            sidx = srcv[pl.ds(i * K + v * L, L)]
            didx = dstv[pl.ds(i * K + v * L, L)]
            dloc = didx - lo
            inr = (dloc >= 0) & (dloc < NH)
            sadj[0, pl.ds(v * L, L)] = jnp.where(inr, sidx, -1)
            dadj[0, pl.ds(v * L, L)] = jnp.where(inr, dloc, -1)
        pltpu.async_copy(
            hw_h.at[plsc.Indices(sadj.at[0], ignored_value=-1)], rows,
            sem).wait()
        for v in range(K // L):
            exvec = exv[pl.ds(i * K + v * L, L)]
            for j in range(L):
                exj = exvec[j]
                r = v * L + j
                for l in range(H // L):
                    rows[r, pl.ds(l * L, L)] = rows[r, pl.ds(l * L, L)] * exj
        pltpu.sync_copy(rows,
                        acc_sh.at[plsc.Indices(dadj.at[0], ignored_value=-1)],
                        add=True)
        return 0

    lax.fori_loop(0, CH2, chunk, 0)
    plsc.subcore_barrier()

    def outstep(k, _):
        cid = s + k * NS

        @pl.when(cid < NRC2)
        def _():
            r0 = cid * L
            pltpu.sync_copy(acc_sh.at[pl.ds(r0, L)], tbuf)
            iv = invv[pl.ds(r0, L)]
            for j in range(L):
                sj = iv[j]
                for l in range(H // L):
                    tbuf[j, pl.ds(l * L, L)] = tbuf[j, pl.ds(l * L, L)] * sj
            pltpu.sync_copy(tbuf, acc_h.at[c, pl.ds(r0, L)])
        return 0

    lax.fori_loop(0, RC2, outstep, 0)


_edge_aggregate = pl.kernel(
    _p2_body,
    out_type=jax.ShapeDtypeStruct((NC, NHP, H), F32),
    mesh=_mesh,
    compiler_params=_sc_params,
    scratch_types=[
        pltpu.VMEM((EPT,), I32),
        pltpu.VMEM((EPT,), I32),
        pltpu.VMEM((EPT,), F32),
        pltpu.VMEM((1, K), I32),
        pltpu.VMEM((1, K), I32),
        pltpu.VMEM((K, H), F32),
        pltpu.VMEM((L, H), F32),
        pltpu.VMEM((L, H), F32),
        pltpu.VMEM((NHP,), F32),
        pltpu.VMEM((NHP,), F32),
        pltpu.VMEM((NHP,), F32),
        pltpu.VMEM_SHARED((NHP, H), F32),
        pltpu.SemaphoreType.DMA,
    ],
)


def _attn_outputs(hw, ats, atd, as_ref, ad_ref, m_ref):
    a_s = jnp.sum(hw * ats, axis=1, keepdims=True)
    a_d = jnp.sum(hw * atd, axis=1, keepdims=True)
    as_ref[...] = a_s
    ad_ref[...] = a_d
    t = jnp.max(a_s) + jnp.max(a_d)
    m_ref[...] = jnp.full((1, H), jnp.maximum(t, 0.2 * t), F32)


def _first_body(x_ref, emb_ref, w_ref, ats_ref, atd_ref,
                hw_ref, as_ref, ad_ref, m_ref):
    oh = (x_ref[...] == lax.broadcasted_iota(I32, (N, V), 1)).astype(F32)
    h = _dot_onehot(oh, emb_ref[...])
    hw = _dot_f32(h, w_ref[...])
    hw_ref[...] = hw
    _attn_outputs(hw, ats_ref[...], atd_ref[...], as_ref, ad_ref, m_ref)


_tc_first = pl.pallas_call(
    _first_body,
    out_shape=[jax.ShapeDtypeStruct((N, H), F32),
               jax.ShapeDtypeStruct((N, 1), F32),
               jax.ShapeDtypeStruct((N, 1), F32),
               jax.ShapeDtypeStruct((1, H), F32)],
)


def _combine_body(acc_ref, b_ref, w_ref, ats_ref, atd_ref,
                  hw_ref, as_ref, ad_ref, m_ref):
    hsum = jnp.concatenate([acc_ref[0, :NH], acc_ref[1, :NH]], axis=0)
    h = jnp.maximum(hsum + b_ref[...], 0.0)
    hw = _dot_f32(h, w_ref[...])
    hw_ref[...] = hw
    _attn_outputs(hw, ats_ref[...], atd_ref[...], as_ref, ad_ref, m_ref)


_tc_combine = pl.pallas_call(
    _combine_body,
    out_shape=[jax.ShapeDtypeStruct((N, H), F32),
               jax.ShapeDtypeStruct((N, 1), F32),
               jax.ShapeDtypeStruct((N, 1), F32),
               jax.ShapeDtypeStruct((1, H), F32)],
)


def _final_body(acc_ref, b_ref, batch_ref, w1_ref, b1_ref, w2_ref, b2_ref,
                y_ref):
    hsum = jnp.concatenate([acc_ref[0, :NH], acc_ref[1, :NH]], axis=0)
    h = jnp.maximum(hsum + b_ref[...], 0.0)
    oh = (lax.broadcasted_iota(I32, (G, N), 0) == batch_ref[...]).astype(F32)
    pool = _dot_onehot(oh, h)
    y1 = jnp.maximum(_dot_f32(pool, w1_ref[...]) + b1_ref[...], 0.0)
    y_ref[...] = _dot_f32(y1, w2_ref[...]) + b2_ref[...]


_tc_final = pl.pallas_call(
    _final_body,
    out_shape=jax.ShapeDtypeStruct((G, 1), F32),
)


def kernel(x, edge_index, batch, eigens, emb, W, att_src, att_dst, bias,
           W1, b1, W2, b2):
    del eigens
    x2 = x.reshape(N, 1).astype(I32)
    src = edge_index[0]
    dst = edge_index[1]
    zeros_n = jnp.zeros((N,), F32)

    hw, a_s, a_d, m = _tc_first(x2, emb, W[0], att_src[0].reshape(1, H),
                                att_dst[0].reshape(1, H))

    roll = [1, 2, 0]
    xs = (bias.reshape(3, 1, H),
          W[jnp.array(roll)],
          att_src[jnp.array(roll)].reshape(3, 1, H),
          att_dst[jnp.array(roll)].reshape(3, 1, H))

    def body(carry, x):
        hw, a_s, a_d, m, _ = carry
        b_i, w_n, ats_n, atd_n = x
        ex, dp0, dp1 = _edge_softmax(src, dst, a_s.reshape(N),
                                     a_d.reshape(N), m, zeros_n)
        acc = _edge_aggregate(hw, src, dst, ex, dp0, dp1)
        hw2, a_s2, a_d2, m2 = _tc_combine(acc, b_i, w_n, ats_n, atd_n)
        return (hw2, a_s2, a_d2, m2, acc), None

    init = (hw, a_s, a_d, m, jnp.zeros((NC, NHP, H), F32))
    (_, _, _, _, acc), _ = lax.scan(body, init, xs, length=3)
    y = _tc_final(acc, bias[2].reshape(1, H),
                  batch.reshape(1, N).astype(I32), W1,
                  b1.reshape(1, H), W2, b2.reshape(1, 1))
    return y.reshape(G)


# speedup vs baseline: 20.5618x; 1.2396x over previous
import jax
import jax.numpy as jnp
from jax import lax
from jax.experimental import pallas as pl
from jax.experimental.pallas import tpu as pltpu
from jax.experimental.pallas import tpu_sc as plsc

N = 10000
E = 320000
H = 128
G = 64
V = 28
NC = 2
NS = 16
NW = NC * NS
EPW = E // NW
K = 80
CH = EPW // K
L = 16
NRC = N // L
RC = (NRC + NS - 1) // NS
EPT = E // NS
CH2 = EPT // K
NH = N // 2
NHP = 5008
NRC2 = NHP // L
RC2 = (NRC2 + NS - 1) // NS
F32 = jnp.float32
I32 = jnp.int32

_HIGH = lax.Precision.HIGHEST
BF16 = jnp.bfloat16


def _split(a):
    hi = a.astype(BF16)
    lo = (a - hi.astype(F32)).astype(BF16)
    return hi, lo


def _dot_f32(a, b):
    return jnp.dot(a, b, preferred_element_type=F32)


def _dot_onehot(oh, b):
    bh, bl = _split(b)
    ohb = oh.astype(BF16)

    def d(x, y):
        return jnp.dot(x, y, preferred_element_type=F32)

    return d(ohb, bh) + d(ohb, bl)

_mesh = plsc.VectorSubcoreMesh(
    core_axis_name="c", subcore_axis_name="s", num_cores=NC, num_subcores=NS)
_sc_params = pltpu.CompilerParams(needs_layout_passes=False)


def _p1_body(src_h, dst_h, as_h, ad_h, m_h, zeros_h, ex_h, dp0_h, dp1_h,
             asv, adv, srcv, dstv, exv, mv, dsh):
    c = lax.axis_index("c")
    s = lax.axis_index("s")
    wid = s * NC + c
    base = wid * EPW

    pltpu.sync_copy(as_h, asv)
    pltpu.sync_copy(ad_h, adv)
    pltpu.sync_copy(src_h.at[pl.ds(base, EPW)], srcv)
    pltpu.sync_copy(dst_h.at[pl.ds(base, EPW)], dstv)

    @pl.when(s == 0)
    def _():
        pltpu.sync_copy(zeros_h, dsh)

    pltpu.sync_copy(m_h.at[0, pl.ds(0, L)], mv)
    m = mv[...]

    def step(i, _):
        sidx = srcv[pl.ds(i * L, L)]
        didx = dstv[pl.ds(i * L, L)]
        t = plsc.load_gather(asv, [sidx]) + plsc.load_gather(adv, [didx])
        e = jnp.maximum(t, 0.2 * t)
        exv[pl.ds(i * L, L)] = jnp.exp(e - m)
        return 0

    lax.fori_loop(0, EPW // L, step, 0)

    plsc.subcore_barrier()
    pltpu.sync_copy(exv, dsh.at[dstv], add=True)
    pltpu.sync_copy(exv, ex_h.at[pl.ds(base, EPW)])
    plsc.subcore_barrier()

    @pl.when((s == 0) & (c == 0))
    def _():
        pltpu.sync_copy(dsh, dp0_h)

    @pl.when((s == 0) & (c == 1))
    def _():
        pltpu.sync_copy(dsh, dp1_h)


_edge_softmax = pl.kernel(
    _p1_body,
    out_type=[jax.ShapeDtypeStruct((E,), F32),
              jax.ShapeDtypeStruct((N,), F32),
              jax.ShapeDtypeStruct((N,), F32)],
    mesh=_mesh,
    compiler_params=_sc_params,
    scratch_types=[
        pltpu.VMEM((N,), F32),
        pltpu.VMEM((N,), F32),
        pltpu.VMEM((EPW,), I32),
        pltpu.VMEM((EPW,), I32),
        pltpu.VMEM((EPW,), F32),
        pltpu.VMEM((L,), F32),
        pltpu.VMEM_SHARED((N,), F32),
    ],
)


def _p2_body(hw_h, pk_h, dp0_h, dp1_h, acc_h,
             pkb0, sadj0, dadj0, rows0, pkb1, sadj1, dadj1, rows1,
             zbuf, tbuf, dp0v, dp1v, invv, acc_sh, sem0, sem1):
    c = lax.axis_index("c")
    s = lax.axis_index("s")
    lo = c * NH

    pltpu.sync_copy(dp0_h.at[pl.ds(lo, NH)], dp0v.at[pl.ds(0, NH)])
    pltpu.sync_copy(dp1_h.at[pl.ds(lo, NH)], dp1v.at[pl.ds(0, NH)])

    def invstep(i, _):
        d = dp0v[pl.ds(i * L, L)] + dp1v[pl.ds(i * L, L)]
        invv[pl.ds(i * L, L)] = 1.0 / (d + 1e-16)
        return 0

    lax.fori_loop(0, NRC2, invstep, 0)

    zv = jnp.zeros((L,), F32)
    for r in range(L):
        for l in range(H // L):
            zbuf[r, pl.ds(l * L, L)] = zv

    def zcpy(k, _):
        cid = s + k * NS

        @pl.when(cid < NRC2)
        def _():
            pltpu.sync_copy(zbuf, acc_sh.at[pl.ds(cid * L, L)])
        return 0

    lax.fori_loop(0, RC2, zcpy, 0)
    plsc.subcore_barrier()

    slots = ((pkb0, sadj0, dadj0, rows0, sem0),
             (pkb1, sadj1, dadj1, rows1, sem1))
    cbase = s * CH2

    def build(i, sl):
        pkb, sa, da, _, _ = sl
        pltpu.sync_copy(pk_h.at[pl.ds((cbase + i) * 3 * K, 3 * K)], pkb)
        for v in range(K // L):
            sidx = pkb[pl.ds(v * L, L)]
            didx = pkb[pl.ds(K + v * L, L)]
            dloc = didx - lo
            inr = (dloc >= 0) & (dloc < NH)
            sa[0, pl.ds(v * L, L)] = jnp.where(inr, sidx, -1)
            da[0, pl.ds(v * L, L)] = jnp.where(inr, dloc, -1)

    def launch(sl):
        pkb, sa, _, rw, se = sl
        pltpu.async_copy(hw_h.at[plsc.Indices(sa.at[0], ignored_value=-1)],
                         rw, se)

    def process(sl):
        pkb, sa, da, rw, se = sl
        pltpu.make_async_copy(
            hw_h.at[plsc.Indices(sa.at[0], ignored_value=-1)], rw, se).wait()
        for v in range(K // L):
            exvec = plsc.bitcast(pkb[pl.ds(2 * K + v * L, L)], F32)
            for j in range(L):
                exj = exvec[j]
                r = v * L + j
                for l in range(H // L):
                    rw[r, pl.ds(l * L, L)] = rw[r, pl.ds(l * L, L)] * exj
        pltpu.sync_copy(rw, acc_sh.at[plsc.Indices(da.at[0],
                                                   ignored_value=-1)],
                        add=True)

    build(0, slots[0])
    launch(slots[0])

    def pair(ii, _):
        build(2 * ii + 1, slots[1])
        launch(slots[1])
        process(slots[0])

        @pl.when(ii + 1 < CH2 // 2)
        def _():
            build(2 * ii + 2, slots[0])
            launch(slots[0])

        process(slots[1])
        return 0

    lax.fori_loop(0, CH2 // 2, pair, 0)
    plsc.subcore_barrier()

    def outstep(k, _):
        cid = s + k * NS

        @pl.when(cid < NRC2)
        def _():
            r0 = cid * L
            pltpu.sync_copy(acc_sh.at[pl.ds(r0, L)], tbuf)
            iv = invv[pl.ds(r0, L)]
            for j in range(L):
                sj = iv[j]
                for l in range(H // L):
                    tbuf[j, pl.ds(l * L, L)] = tbuf[j, pl.ds(l * L, L)] * sj
            pltpu.sync_copy(tbuf, acc_h.at[c, pl.ds(r0, L)])
        return 0

    lax.fori_loop(0, RC2, outstep, 0)


_edge_aggregate = pl.kernel(
    _p2_body,
    out_type=jax.ShapeDtypeStruct((NC, NHP, H), F32),
    mesh=_mesh,
    compiler_params=_sc_params,
    scratch_types=[
        pltpu.VMEM((3 * K,), I32),
        pltpu.VMEM((1, K), I32),
        pltpu.VMEM((1, K), I32),
        pltpu.VMEM((K, H), F32),
        pltpu.VMEM((3 * K,), I32),
        pltpu.VMEM((1, K), I32),
        pltpu.VMEM((1, K), I32),
        pltpu.VMEM((K, H), F32),
        pltpu.VMEM((L, H), F32),
        pltpu.VMEM((L, H), F32),
        pltpu.VMEM((NHP,), F32),
        pltpu.VMEM((NHP,), F32),
        pltpu.VMEM((NHP,), F32),
        pltpu.VMEM_SHARED((NHP, H), F32),
        pltpu.SemaphoreType.DMA,
        pltpu.SemaphoreType.DMA,
    ],
)


def _attn_outputs(hw, ats, atd, as_ref, ad_ref, m_ref):
    a_s = jnp.sum(hw * ats, axis=1, keepdims=True)
    a_d = jnp.sum(hw * atd, axis=1, keepdims=True)
    as_ref[...] = a_s
    ad_ref[...] = a_d
    t = jnp.max(a_s) + jnp.max(a_d)
    m_ref[...] = jnp.full((1, H), jnp.maximum(t, 0.2 * t), F32)


def _first_body(x_ref, emb_ref, w_ref, ats_ref, atd_ref,
                hw_ref, as_ref, ad_ref, m_ref):
    oh = (x_ref[...] == lax.broadcasted_iota(I32, (N, V), 1)).astype(F32)
    h = _dot_onehot(oh, emb_ref[...])
    hw = _dot_f32(h, w_ref[...])
    hw_ref[...] = hw
    _attn_outputs(hw, ats_ref[...], atd_ref[...], as_ref, ad_ref, m_ref)


_tc_first = pl.pallas_call(
    _first_body,
    out_shape=[jax.ShapeDtypeStruct((N, H), F32),
               jax.ShapeDtypeStruct((N, 1), F32),
               jax.ShapeDtypeStruct((N, 1), F32),
               jax.ShapeDtypeStruct((1, H), F32)],
)


def _combine_body(acc_ref, b_ref, w_ref, ats_ref, atd_ref,
                  hw_ref, as_ref, ad_ref, m_ref):
    hsum = jnp.concatenate([acc_ref[0, :NH], acc_ref[1, :NH]], axis=0)
    h = jnp.maximum(hsum + b_ref[...], 0.0)
    hw = _dot_f32(h, w_ref[...])
    hw_ref[...] = hw
    _attn_outputs(hw, ats_ref[...], atd_ref[...], as_ref, ad_ref, m_ref)


_tc_combine = pl.pallas_call(
    _combine_body,
    out_shape=[jax.ShapeDtypeStruct((N, H), F32),
               jax.ShapeDtypeStruct((N, 1), F32),
               jax.ShapeDtypeStruct((N, 1), F32),
               jax.ShapeDtypeStruct((1, H), F32)],
)


def _final_body(acc_ref, b_ref, batch_ref, w1_ref, b1_ref, w2_ref, b2_ref,
                y_ref):
    hsum = jnp.concatenate([acc_ref[0, :NH], acc_ref[1, :NH]], axis=0)
    h = jnp.maximum(hsum + b_ref[...], 0.0)
    oh = (lax.broadcasted_iota(I32, (G, N), 0) == batch_ref[...]).astype(F32)
    pool = _dot_onehot(oh, h)
    y1 = jnp.maximum(_dot_f32(pool, w1_ref[...]) + b1_ref[...], 0.0)
    y_ref[...] = _dot_f32(y1, w2_ref[...]) + b2_ref[...]


_tc_final = pl.pallas_call(
    _final_body,
    out_shape=jax.ShapeDtypeStruct((G, 1), F32),
)


def kernel(x, edge_index, batch, eigens, emb, W, att_src, att_dst, bias,
           W1, b1, W2, b2):
    del eigens
    x2 = x.reshape(N, 1).astype(I32)
    src = edge_index[0]
    dst = edge_index[1]
    zeros_n = jnp.zeros((N,), F32)

    hw, a_s, a_d, m = _tc_first(x2, emb, W[0], att_src[0].reshape(1, H),
                                att_dst[0].reshape(1, H))

    roll = [1, 2, 0]
    xs = (bias.reshape(3, 1, H),
          W[jnp.array(roll)],
          att_src[jnp.array(roll)].reshape(3, 1, H),
          att_dst[jnp.array(roll)].reshape(3, 1, H))

    def body(carry, x):
        hw, a_s, a_d, m, _ = carry
        b_i, w_n, ats_n, atd_n = x
        ex, dp0, dp1 = _edge_softmax(src, dst, a_s.reshape(N),
                                     a_d.reshape(N), m, zeros_n)
        pk = jnp.concatenate(
            [src.reshape(-1, K), dst.reshape(-1, K),
             lax.bitcast_convert_type(ex, I32).reshape(-1, K)],
            axis=1).reshape(-1)
        acc = _edge_aggregate(hw, pk, dp0, dp1)
        hw2, a_s2, a_d2, m2 = _tc_combine(acc, b_i, w_n, ats_n, atd_n)
        return (hw2, a_s2, a_d2, m2, acc), None

    init = (hw, a_s, a_d, m, jnp.zeros((NC, NHP, H), F32))
    (_, _, _, _, acc), _ = lax.scan(body, init, xs, length=3)
    y = _tc_final(acc, bias[2].reshape(1, H),
                  batch.reshape(1, N).astype(I32), W1,
                  b1.reshape(1, H), W2, b2.reshape(1, 1))
    return y.reshape(G)


# speedup vs baseline: 20.5720x; 1.0005x over previous
import jax
import jax.numpy as jnp
from jax import lax
from jax.experimental import pallas as pl
from jax.experimental.pallas import tpu as pltpu
from jax.experimental.pallas import tpu_sc as plsc

N = 10000
E = 320000
H = 128
G = 64
V = 28
NC = 2
NS = 16
NW = NC * NS
EPW = E // NW
K = 80
CH = EPW // K
L = 16
NRC = N // L
RC = (NRC + NS - 1) // NS
EPT = E // NS
CH2 = EPT // K
NH = N // 2
NHP = 5008
NRC2 = NHP // L
RC2 = (NRC2 + NS - 1) // NS
F32 = jnp.float32
I32 = jnp.int32

_HIGH = lax.Precision.HIGHEST
BF16 = jnp.bfloat16


def _split(a):
    hi = a.astype(BF16)
    lo = (a - hi.astype(F32)).astype(BF16)
    return hi, lo


def _dot_f32(a, b):
    return jnp.dot(a, b, preferred_element_type=F32)


def _dot_onehot(oh, b):
    bh, bl = _split(b)
    ohb = oh.astype(BF16)

    def d(x, y):
        return jnp.dot(x, y, preferred_element_type=F32)

    return d(ohb, bh) + d(ohb, bl)

_mesh = plsc.VectorSubcoreMesh(
    core_axis_name="c", subcore_axis_name="s", num_cores=NC, num_subcores=NS)
_sc_params = pltpu.CompilerParams(needs_layout_passes=False)


def _p1_body(src_h, dst_h, as_h, ad_h, m_h, zeros_h, ex_h, dp0_h, dp1_h,
             asv, adv, srcv, dstv, exv, mv, dsh):
    c = lax.axis_index("c")
    s = lax.axis_index("s")
    wid = s * NC + c
    base = wid * EPW

    pltpu.sync_copy(as_h, asv)
    pltpu.sync_copy(ad_h, adv)
    pltpu.sync_copy(src_h.at[pl.ds(base, EPW)], srcv)
    pltpu.sync_copy(dst_h.at[pl.ds(base, EPW)], dstv)

    @pl.when(s == 0)
    def _():
        pltpu.sync_copy(zeros_h, dsh)

    pltpu.sync_copy(m_h.at[0, pl.ds(0, L)], mv)
    m = mv[...]

    def step(i, _):
        sidx = srcv[pl.ds(i * L, L)]
        didx = dstv[pl.ds(i * L, L)]
        t = plsc.load_gather(asv, [sidx]) + plsc.load_gather(adv, [didx])
        e = jnp.maximum(t, 0.2 * t)
        exv[pl.ds(i * L, L)] = jnp.exp(e - m)
        return 0

    lax.fori_loop(0, EPW // L, step, 0)

    plsc.subcore_barrier()
    pltpu.sync_copy(exv, dsh.at[dstv], add=True)
    pltpu.sync_copy(exv, ex_h.at[pl.ds(base, EPW)])
    plsc.subcore_barrier()

    @pl.when((s == 0) & (c == 0))
    def _():
        pltpu.sync_copy(dsh, dp0_h)

    @pl.when((s == 0) & (c == 1))
    def _():
        pltpu.sync_copy(dsh, dp1_h)


_edge_softmax = pl.kernel(
    _p1_body,
    out_type=[jax.ShapeDtypeStruct((E,), F32),
              jax.ShapeDtypeStruct((N,), F32),
              jax.ShapeDtypeStruct((N,), F32)],
    mesh=_mesh,
    compiler_params=_sc_params,
    scratch_types=[
        pltpu.VMEM((N,), F32),
        pltpu.VMEM((N,), F32),
        pltpu.VMEM((EPW,), I32),
        pltpu.VMEM((EPW,), I32),
        pltpu.VMEM((EPW,), F32),
        pltpu.VMEM((L,), F32),
        pltpu.VMEM_SHARED((N,), F32),
    ],
)


def _p2_body(hw_h, pk_h, dp0_h, dp1_h, acc_h,
             pkb0, sadj0, dadj0, rows0, pkb1, sadj1, dadj1, rows1,
             zbuf, tbuf, dp0v, dp1v, invv, acc_sh, sem0, sem1, ssem0, ssem1):
    c = lax.axis_index("c")
    s = lax.axis_index("s")
    lo = c * NH

    pltpu.sync_copy(dp0_h.at[pl.ds(lo, NH)], dp0v.at[pl.ds(0, NH)])
    pltpu.sync_copy(dp1_h.at[pl.ds(lo, NH)], dp1v.at[pl.ds(0, NH)])

    def invstep(i, _):
        d = dp0v[pl.ds(i * L, L)] + dp1v[pl.ds(i * L, L)]
        invv[pl.ds(i * L, L)] = 1.0 / (d + 1e-16)
        return 0

    lax.fori_loop(0, NRC2, invstep, 0)

    zv = jnp.zeros((L,), F32)
    for r in range(L):
        for l in range(H // L):
            zbuf[r, pl.ds(l * L, L)] = zv

    def zcpy(k, _):
        cid = s + k * NS

        @pl.when(cid < NRC2)
        def _():
            pltpu.sync_copy(zbuf, acc_sh.at[pl.ds(cid * L, L)])
        return 0

    lax.fori_loop(0, RC2, zcpy, 0)
    plsc.subcore_barrier()

    slots = ((pkb0, sadj0, dadj0, rows0, sem0, ssem0),
             (pkb1, sadj1, dadj1, rows1, sem1, ssem1))
    cbase = s * CH2

    def drain(sl):
        _, _, da, rw, _, sse = sl
        pltpu.make_async_copy(
            rw, acc_sh.at[plsc.Indices(da.at[0], ignored_value=-1)],
            sse).wait()

    def build(i, sl):
        pkb, sa, da, rw, _, sse = sl

        @pl.when(i >= 2)
        def _():
            drain(sl)

        pltpu.sync_copy(pk_h.at[pl.ds((cbase + i) * 3 * K, 3 * K)], pkb)
        for v in range(K // L):
            sidx = pkb[pl.ds(v * L, L)]
            didx = pkb[pl.ds(K + v * L, L)]
            dloc = didx - lo
            inr = (dloc >= 0) & (dloc < NH)
            sa[0, pl.ds(v * L, L)] = jnp.where(inr, sidx, -1)
            da[0, pl.ds(v * L, L)] = jnp.where(inr, dloc, -1)

    def launch(sl):
        pkb, sa, _, rw, se, _ = sl
        pltpu.async_copy(hw_h.at[plsc.Indices(sa.at[0], ignored_value=-1)],
                         rw, se)

    def process(sl):
        pkb, sa, da, rw, se, sse = sl
        pltpu.make_async_copy(
            hw_h.at[plsc.Indices(sa.at[0], ignored_value=-1)], rw, se).wait()
        for v in range(K // L):
            exvec = plsc.bitcast(pkb[pl.ds(2 * K + v * L, L)], F32)
            for j in range(L):
                exj = exvec[j]
                r = v * L + j
                for l in range(H // L):
                    rw[r, pl.ds(l * L, L)] = rw[r, pl.ds(l * L, L)] * exj
        pltpu.async_copy(rw,
                         acc_sh.at[plsc.Indices(da.at[0], ignored_value=-1)],
                         sse, add=True)

    build(0, slots[0])
    launch(slots[0])

    def pair(ii, _):
        build(2 * ii + 1, slots[1])
        launch(slots[1])
        process(slots[0])

        @pl.when(ii + 1 < CH2 // 2)
        def _():
            build(2 * ii + 2, slots[0])
            launch(slots[0])

        process(slots[1])
        return 0

    lax.fori_loop(0, CH2 // 2, pair, 0)
    drain(slots[0])
    drain(slots[1])
    plsc.subcore_barrier()

    def outstep(k, _):
        cid = s + k * NS

        @pl.when(cid < NRC2)
        def _():
            r0 = cid * L
            pltpu.sync_copy(acc_sh.at[pl.ds(r0, L)], tbuf)
            iv = invv[pl.ds(r0, L)]
            for j in range(L):
                sj = iv[j]
                for l in range(H // L):
                    tbuf[j, pl.ds(l * L, L)] = tbuf[j, pl.ds(l * L, L)] * sj
            pltpu.sync_copy(tbuf, acc_h.at[c, pl.ds(r0, L)])
        return 0

    lax.fori_loop(0, RC2, outstep, 0)


_edge_aggregate = pl.kernel(
    _p2_body,
    out_type=jax.ShapeDtypeStruct((NC, NHP, H), F32),
    mesh=_mesh,
    compiler_params=_sc_params,
    scratch_types=[
        pltpu.VMEM((3 * K,), I32),
        pltpu.VMEM((1, K), I32),
        pltpu.VMEM((1, K), I32),
        pltpu.VMEM((K, H), F32),
        pltpu.VMEM((3 * K,), I32),
        pltpu.VMEM((1, K), I32),
        pltpu.VMEM((1, K), I32),
        pltpu.VMEM((K, H), F32),
        pltpu.VMEM((L, H), F32),
        pltpu.VMEM((L, H), F32),
        pltpu.VMEM((NHP,), F32),
        pltpu.VMEM((NHP,), F32),
        pltpu.VMEM((NHP,), F32),
        pltpu.VMEM_SHARED((NHP, H), F32),
        pltpu.SemaphoreType.DMA,
        pltpu.SemaphoreType.DMA,
        pltpu.SemaphoreType.DMA,
        pltpu.SemaphoreType.DMA,
    ],
)


def _attn_outputs(hw, ats, atd, as_ref, ad_ref, m_ref):
    a_s = jnp.sum(hw * ats, axis=1, keepdims=True)
    a_d = jnp.sum(hw * atd, axis=1, keepdims=True)
    as_ref[...] = a_s
    ad_ref[...] = a_d
    t = jnp.max(a_s) + jnp.max(a_d)
    m_ref[...] = jnp.full((1, H), jnp.maximum(t, 0.2 * t), F32)


def _first_body(x_ref, emb_ref, w_ref, ats_ref, atd_ref,
                hw_ref, as_ref, ad_ref, m_ref):
    oh = (x_ref[...] == lax.broadcasted_iota(I32, (N, V), 1)).astype(F32)
    h = _dot_onehot(oh, emb_ref[...])
    hw = _dot_f32(h, w_ref[...])
    hw_ref[...] = hw
    _attn_outputs(hw, ats_ref[...], atd_ref[...], as_ref, ad_ref, m_ref)


_tc_first = pl.pallas_call(
    _first_body,
    out_shape=[jax.ShapeDtypeStruct((N, H), F32),
               jax.ShapeDtypeStruct((N, 1), F32),
               jax.ShapeDtypeStruct((N, 1), F32),
               jax.ShapeDtypeStruct((1, H), F32)],
)


def _combine_body(acc_ref, b_ref, w_ref, ats_ref, atd_ref,
                  hw_ref, as_ref, ad_ref, m_ref):
    hsum = jnp.concatenate([acc_ref[0, :NH], acc_ref[1, :NH]], axis=0)
    h = jnp.maximum(hsum + b_ref[...], 0.0)
    hw = _dot_f32(h, w_ref[...])
    hw_ref[...] = hw
    _attn_outputs(hw, ats_ref[...], atd_ref[...], as_ref, ad_ref, m_ref)


_tc_combine = pl.pallas_call(
    _combine_body,
    out_shape=[jax.ShapeDtypeStruct((N, H), F32),
               jax.ShapeDtypeStruct((N, 1), F32),
               jax.ShapeDtypeStruct((N, 1), F32),
               jax.ShapeDtypeStruct((1, H), F32)],
)


def _final_body(acc_ref, b_ref, batch_ref, w1_ref, b1_ref, w2_ref, b2_ref,
                y_ref):
    hsum = jnp.concatenate([acc_ref[0, :NH], acc_ref[1, :NH]], axis=0)
    h = jnp.maximum(hsum + b_ref[...], 0.0)
    oh = (lax.broadcasted_iota(I32, (G, N), 0) == batch_ref[...]).astype(F32)
    pool = _dot_onehot(oh, h)
    y1 = jnp.maximum(_dot_f32(pool, w1_ref[...]) + b1_ref[...], 0.0)
    y_ref[...] = _dot_f32(y1, w2_ref[...]) + b2_ref[...]


_tc_final = pl.pallas_call(
    _final_body,
    out_shape=jax.ShapeDtypeStruct((G, 1), F32),
)


def kernel(x, edge_index, batch, eigens, emb, W, att_src, att_dst, bias,
           W1, b1, W2, b2):
    del eigens
    x2 = x.reshape(N, 1).astype(I32)
    src = edge_index[0]
    dst = edge_index[1]
    zeros_n = jnp.zeros((N,), F32)

    hw, a_s, a_d, m = _tc_first(x2, emb, W[0], att_src[0].reshape(1, H),
                                att_dst[0].reshape(1, H))

    roll = [1, 2, 0]
    xs = (bias.reshape(3, 1, H),
          W[jnp.array(roll)],
          att_src[jnp.array(roll)].reshape(3, 1, H),
          att_dst[jnp.array(roll)].reshape(3, 1, H))

    def body(carry, x):
        hw, a_s, a_d, m, _ = carry
        b_i, w_n, ats_n, atd_n = x
        ex, dp0, dp1 = _edge_softmax(src, dst, a_s.reshape(N),
                                     a_d.reshape(N), m, zeros_n)
        pk = jnp.concatenate(
            [src.reshape(-1, K), dst.reshape(-1, K),
             lax.bitcast_convert_type(ex, I32).reshape(-1, K)],
            axis=1).reshape(-1)
        acc = _edge_aggregate(hw, pk, dp0, dp1)
        hw2, a_s2, a_d2, m2 = _tc_combine(acc, b_i, w_n, ats_n, atd_n)
        return (hw2, a_s2, a_d2, m2, acc), None

    init = (hw, a_s, a_d, m, jnp.zeros((NC, NHP, H), F32))
    (_, _, _, _, acc), _ = lax.scan(body, init, xs, length=3)
    y = _tc_final(acc, bias[2].reshape(1, H),
                  batch.reshape(1, N).astype(I32), W1,
                  b1.reshape(1, H), W2, b2.reshape(1, 1))
    return y.reshape(G)
